# baseline (speedup 1.0000x reference)
import jax
import jax.numpy as jnp
from jax.experimental import pallas as pl
from jax.experimental.pallas import tpu as pltpu

B, N, D = 32, 576, 768
ROWS = B * N
NCHUNK = 8
CHUNK = ROWS // NCHUNK


def _body(idx_ref, embed_ref, tok_hbm, out_hbm, buf, in_sem, out_sem):
    row = embed_ref[pl.ds(idx_ref[0], 1), :]
    loads = [
        pltpu.make_async_copy(
            tok_hbm.at[pl.ds(k * CHUNK, CHUNK)], buf.at[k], in_sem.at[k])
        for k in range(NCHUNK)
    ]
    stores = [
        pltpu.make_async_copy(
            buf.at[k], out_hbm.at[pl.ds(k * CHUNK, CHUNK)], out_sem.at[k])
        for k in range(NCHUNK)
    ]
    for ld in loads:
        ld.start()
    for k in range(NCHUNK):
        loads[k].wait()
        buf[k] = buf[k] + row
        stores[k].start()
    for st in stores:
        st.wait()


def kernel(tokens, embed, frame_idx):
    idx = jnp.asarray(frame_idx, dtype=jnp.int32).reshape((1,))
    tok2 = tokens.reshape(ROWS, D)
    out = pl.pallas_call(
        _body,
        in_specs=[
            pl.BlockSpec(memory_space=pltpu.MemorySpace.SMEM),
            pl.BlockSpec(memory_space=pltpu.MemorySpace.VMEM),
            pl.BlockSpec(memory_space=pltpu.MemorySpace.HBM),
        ],
        out_specs=pl.BlockSpec(memory_space=pltpu.MemorySpace.HBM),
        out_shape=jax.ShapeDtypeStruct((ROWS, D), tokens.dtype),
        scratch_shapes=[
            pltpu.VMEM((NCHUNK, CHUNK, D), jnp.float32),
            pltpu.SemaphoreType.DMA((NCHUNK,)),
            pltpu.SemaphoreType.DMA((NCHUNK,)),
        ],
        compiler_params=pltpu.CompilerParams(
            vmem_limit_bytes=60 * 1024 * 1024,
        ),
    )(idx, embed, tok2)
    return out.reshape(B, N, D)
